# baseline (speedup 1.0000x reference)
import functools

import jax
import jax.numpy as jnp
from jax import lax
from jax.experimental import pallas as pl
from jax.experimental.pallas import tpu as pltpu
from jax.experimental.pallas import tpu_sc as plsc

B = 64
DOC = 4096
NOPT = 16
NWORD = 4
TABLE = 50048
NC, NS, L = 2, 16, 16
NW = NC * NS
B_PER_W = B // NW


def _sc_kernel(doc_idx_hbm, doc_prob_hbm, opt_hbm, out_hbm,
               table_v, table2_v, idx_v, prob_v, opt_v, out_v,
               sem_opt, sem_b0, sem_b1):
    wid = lax.axis_index("s") * NC + lax.axis_index("c")
    b0 = wid * B_PER_W
    sems = [sem_b0, sem_b1]
    cp_opt = pltpu.make_async_copy(opt_hbm.at[pl.ds(b0, B_PER_W)], opt_v, sem_opt)
    cp_opt.start()
    copies = []
    for bi in range(B_PER_W):
        ci = pltpu.make_async_copy(doc_idx_hbm.at[b0 + bi], idx_v.at[bi], sems[bi])
        cp = pltpu.make_async_copy(doc_prob_hbm.at[b0 + bi], prob_v.at[bi], sems[bi])
        ci.start()
        cp.start()
        copies.append((ci, cp))

    zeros = jnp.zeros((L,), jnp.float32)
    one = jnp.ones((L,), jnp.float32)
    lanes4 = lax.iota(jnp.int32, L) * NWORD
    tables = [table_v, table2_v]

    cp_opt.wait()
    opt_vecs = [[], []]
    for bi in range(B_PER_W):
        bvec = jnp.full((L,), bi, jnp.int32)
        for w in range(NWORD):
            ow = plsc.load_gather(opt_v, [bvec, lanes4 + w])
            opt_vecs[bi].append(ow)
            plsc.store_scatter(tables[bi], [ow], zeros)

    for bi in range(B_PER_W):
        copies[bi][0].wait()
        copies[bi][1].wait()

    def body(i):
        for bi in range(B_PER_W):
            di = idx_v[bi, pl.ds(i * L, L)]
            pv = prob_v[bi, pl.ds(i * L, L)]
            plsc.addupdate_scatter(tables[bi], [di], pv)

    plsc.parallel_loop(0, DOC // L, 1, unroll=4)(body)

    for bi in range(B_PER_W):
        num = zeros
        den = zeros
        for w in range(NWORD):
            ow = opt_vecs[bi][w]
            num = num + plsc.load_gather(tables[bi], [ow])
            den = den + jnp.where(ow != 0, one, zeros)
        out_v[bi, :] = num / den
    pltpu.sync_copy(out_v, out_hbm.at[pl.ds(b0, B_PER_W)])


def kernel(document_indices, document_probabilities, options):
    opt_q = options.reshape(B, NOPT * NWORD)
    mesh = plsc.VectorSubcoreMesh(core_axis_name="c", subcore_axis_name="s",
                                  num_cores=NC, num_subcores=NS)
    run = functools.partial(
        pl.kernel,
        out_type=jax.ShapeDtypeStruct((B, NOPT), jnp.float32),
        mesh=mesh,
        compiler_params=pltpu.CompilerParams(
            needs_layout_passes=False,
            skip_device_barrier=True,
            disable_bounds_checks=True,
            disable_semaphore_checks=True,
        ),
        scratch_types=[
            pltpu.VMEM((TABLE,), jnp.float32),
            pltpu.VMEM((TABLE,), jnp.float32),
            pltpu.VMEM((B_PER_W, DOC), jnp.int32),
            pltpu.VMEM((B_PER_W, DOC), jnp.float32),
            pltpu.VMEM((B_PER_W, NOPT * NWORD), jnp.int32),
            pltpu.VMEM((B_PER_W, NOPT), jnp.float32),
            pltpu.SemaphoreType.DMA,
            pltpu.SemaphoreType.DMA,
            pltpu.SemaphoreType.DMA,
        ],
    )(_sc_kernel)
    return run(document_indices, document_probabilities, opt_q)


# speedup vs baseline: 1.0169x; 1.0169x over previous
import functools

import jax
import jax.numpy as jnp
from jax import lax
from jax.experimental import pallas as pl
from jax.experimental.pallas import tpu as pltpu
from jax.experimental.pallas import tpu_sc as plsc

B = 64
DOC = 4096
NOPT = 16
NWORD = 4
TABLE = 50048
NC, NS, L = 2, 16, 16
NW = NC * NS
B_PER_W = B // NW


def _sc_kernel(doc_idx_hbm, doc_prob_hbm, opt_hbm, out_hbm,
               table_v, table2_v, idx_v, prob_v, opt_v, out_v,
               sem_opt, sem_b0, sem_b1):
    wid = lax.axis_index("s") * NC + lax.axis_index("c")
    b0 = wid * B_PER_W
    sems = [sem_b0, sem_b1]
    cp_opt = pltpu.make_async_copy(opt_hbm.at[pl.ds(b0, B_PER_W)], opt_v, sem_opt)
    cp_opt.start()
    copies = []
    for bi in range(B_PER_W):
        ci = pltpu.make_async_copy(doc_idx_hbm.at[b0 + bi], idx_v.at[bi], sems[bi])
        cp = pltpu.make_async_copy(doc_prob_hbm.at[b0 + bi], prob_v.at[bi], sems[bi])
        ci.start()
        cp.start()
        copies.append((ci, cp))

    zeros = jnp.zeros((L,), jnp.float32)
    one = jnp.ones((L,), jnp.float32)
    lanes4 = lax.iota(jnp.int32, L) * NWORD
    tables = [table_v, table2_v]

    cp_opt.wait()
    opt_vecs = [[], []]
    for bi in range(B_PER_W):
        bvec = jnp.full((L,), bi, jnp.int32)
        for w in range(NWORD):
            ow = plsc.load_gather(opt_v, [bvec, lanes4 + w])
            opt_vecs[bi].append(ow)
            plsc.store_scatter(tables[bi], [ow], zeros)

    for bi in range(B_PER_W):
        copies[bi][0].wait()
        copies[bi][1].wait()

        def body(i, bi=bi):
            di = idx_v[bi, pl.ds(i * L, L)]
            pv = prob_v[bi, pl.ds(i * L, L)]
            plsc.addupdate_scatter(tables[bi], [di], pv)

        plsc.parallel_loop(0, DOC // L, 1, unroll=8)(body)

    for bi in range(B_PER_W):
        num = zeros
        den = zeros
        for w in range(NWORD):
            ow = opt_vecs[bi][w]
            num = num + plsc.load_gather(tables[bi], [ow])
            den = den + jnp.where(ow != 0, one, zeros)
        out_v[bi, :] = num / den
    pltpu.sync_copy(out_v, out_hbm.at[pl.ds(b0, B_PER_W)])


def kernel(document_indices, document_probabilities, options):
    opt_q = options.reshape(B, NOPT * NWORD)
    mesh = plsc.VectorSubcoreMesh(core_axis_name="c", subcore_axis_name="s",
                                  num_cores=NC, num_subcores=NS)
    run = functools.partial(
        pl.kernel,
        out_type=jax.ShapeDtypeStruct((B, NOPT), jnp.float32),
        mesh=mesh,
        compiler_params=pltpu.CompilerParams(
            needs_layout_passes=False,
            skip_device_barrier=True,
            disable_bounds_checks=True,
            disable_semaphore_checks=True,
        ),
        scratch_types=[
            pltpu.VMEM((TABLE,), jnp.float32),
            pltpu.VMEM((TABLE,), jnp.float32),
            pltpu.VMEM((B_PER_W, DOC), jnp.int32),
            pltpu.VMEM((B_PER_W, DOC), jnp.float32),
            pltpu.VMEM((B_PER_W, NOPT * NWORD), jnp.int32),
            pltpu.VMEM((B_PER_W, NOPT), jnp.float32),
            pltpu.SemaphoreType.DMA,
            pltpu.SemaphoreType.DMA,
            pltpu.SemaphoreType.DMA,
        ],
    )(_sc_kernel)
    return run(document_indices, document_probabilities, opt_q)
